# baseline (speedup 1.0000x reference)
import jax
import jax.numpy as jnp
from jax import lax
from jax.experimental import pallas as pl
from jax.experimental.pallas import tpu as pltpu
from jax.experimental.pallas import tpu_sc as plsc

D_FIELD = 32
BLK = 128
UNITS = 4
CHUNK = UNITS * BLK


def _gather_kernel(batch, hist, num_cores, num_subcores):
    nw = num_cores * num_subcores
    n_blocks = batch // BLK
    blocks_per_t = n_blocks // UNITS
    n_chunks_total = hist * blocks_per_t
    n = n_chunks_total // nw
    assert batch % (BLK * UNITS) == 0 and n_chunks_total % nw == 0
    assert n % 2 == 0
    mesh = plsc.VectorSubcoreMesh(core_axis_name="c", subcore_axis_name="s")

    def body(lookup_hbm, table_hbm, out_hbm, idx_v, rows_v, ptile_v, *sems):
        sem_i = sems[0:2]
        sem_g = sems[2:4]
        sem_o = sems[4:6]
        wid = lax.axis_index("s") * num_cores + lax.axis_index("c")
        c0 = wid * n
        lane = jax.lax.iota(jnp.int32, 16)

        def coords(i):
            cg = c0 + i
            return cg // blocks_per_t, (cg % blocks_per_t) * UNITS

        def a_copy(i, b):
            t, bc0 = coords(i)
            return pltpu.make_async_copy(
                lookup_hbm.at[t, pl.ds(bc0 * BLK, CHUNK)], idx_v.at[b],
                sem_i[b])

        def b_copy(b):
            return pltpu.make_async_copy(
                table_hbm.at[idx_v.at[b]], rows_v.at[b], sem_g[b])

        def d_copies(i, b):
            t, bc0 = coords(i)
            return [
                pltpu.make_async_copy(
                    ptile_v.at[b, dr],
                    out_hbm.at[t, dr, pl.ds(bc0, UNITS)], sem_o[b])
                for dr in range(4)
            ]

        def transpose(b):
            def j_body(j, carry):
                jbase = j * BLK
                for g in range(8):
                    row_vec = lane + (jbase + g * 16)
                    for d in range(D_FIELD):
                        col_vec = jnp.full((16,), d, jnp.int32)
                        v = plsc.load_gather(rows_v.at[b], [row_vec, col_vec])
                        ptile_v[b, d // 8, j, d % 8, pl.ds(g * 16, 16)] = v
                return carry

            lax.fori_loop(0, UNITS, j_body, 0)

        def chunk_pair(p, carry):
            for b in range(2):
                i = 2 * p + b
                b_copy(b).wait()
                @pl.when(i + 1 < n)
                def _():
                    a_copy(0, 1 - b).wait()
                    b_copy(1 - b).start()
                @pl.when(i >= 2)
                def _():
                    for d in d_copies(0, b):
                        d.wait()
                transpose(b)
                for d in d_copies(i, b):
                    d.start()
                @pl.when(i + 2 < n)
                def _():
                    a_copy(i + 2, b).start()
            return carry

        a_copy(0, 0).start()
        a_copy(1, 1).start()
        a_copy(0, 0).wait()
        b_copy(0).start()

        lax.fori_loop(0, n // 2, chunk_pair, 0)

        for b in range(2):
            for d in d_copies(0, b):
                d.wait()

    return body, mesh


def kernel(lookup, table):
    batch, hist = lookup.shape
    info = plsc.get_sparse_core_info()
    body, mesh = _gather_kernel(batch, hist, info.num_cores,
                                info.num_subcores)
    p = pl.kernel(
        body,
        out_type=jax.ShapeDtypeStruct(
            (hist, 4, batch // BLK, 8, BLK), jnp.float32),
        mesh=mesh,
        scratch_types=[
            pltpu.VMEM((2, CHUNK), jnp.int32),
            pltpu.VMEM((2, CHUNK, D_FIELD), jnp.float32),
            pltpu.VMEM((2, 4, UNITS, 8, BLK), jnp.float32),
        ] + [pltpu.SemaphoreType.DMA] * 6,
        compiler_params=pltpu.CompilerParams(use_tc_tiling_on_sc=False,
                                             needs_layout_passes=False),
    )(lookup.T.astype(jnp.int32), table)
    q = p.transpose(2, 4, 0, 1, 3)
    return q.reshape(batch, hist, D_FIELD)


# speedup vs baseline: 1.2753x; 1.2753x over previous
import jax
import jax.numpy as jnp
from jax import lax
from jax.experimental import pallas as pl
from jax.experimental.pallas import tpu as pltpu
from jax.experimental.pallas import tpu_sc as plsc

D_FIELD = 32
BLK = 128
UNITS = 4
CHUNK = UNITS * BLK


def _gather_kernel(batch, hist, num_cores, num_subcores):
    nw = num_cores * num_subcores
    n_blocks = batch // BLK
    blocks_per_t = n_blocks // UNITS
    n_chunks_total = hist * blocks_per_t
    n = n_chunks_total // nw
    assert batch % (BLK * UNITS) == 0 and n_chunks_total % nw == 0
    assert n % 2 == 0
    mesh = plsc.VectorSubcoreMesh(core_axis_name="c", subcore_axis_name="s")

    def body(lookup_hbm, table_hbm, out_hbm, idx_v, rows_v, pad_v, ptile_v,
             *sems):
        sem_i = sems[0:2]
        sem_g = sems[2:4]
        sem_o = sems[4:6]
        wid = lax.axis_index("s") * num_cores + lax.axis_index("c")
        c0 = wid * n
        lane = jax.lax.iota(jnp.int32, 16)

        def coords(i):
            cg = c0 + i
            return cg // blocks_per_t, (cg % blocks_per_t) * UNITS

        def a_copy(i, b):
            t, bc0 = coords(i)
            return pltpu.make_async_copy(
                lookup_hbm.at[t, pl.ds(bc0 * BLK, CHUNK)], idx_v.at[b],
                sem_i[b])

        def b_copy(b):
            return pltpu.make_async_copy(
                table_hbm.at[idx_v.at[b]], rows_v.at[b], sem_g[b])

        def d_copies(i, b):
            t, bc0 = coords(i)
            return [
                pltpu.make_async_copy(
                    ptile_v.at[b, dr],
                    out_hbm.at[t, dr, pl.ds(bc0, UNITS)], sem_o[b])
                for dr in range(4)
            ]

        def transpose(b):
            def repitch(rr, carry):
                for k in range(8):
                    r = rr * 8 + k
                    pad_v[r, pl.ds(0, 16)] = rows_v[b, r, pl.ds(0, 16)]
                    pad_v[r, pl.ds(16, 16)] = rows_v[b, r, pl.ds(16, 16)]
                return carry

            lax.fori_loop(0, CHUNK // 8, repitch, 0)

            def g_body(g, carry):
                for j in range(UNITS):
                    row_vec = lane + (j * BLK + g * 16)
                    for d in range(D_FIELD):
                        col_vec = jnp.full((16,), d, jnp.int32)
                        v = plsc.load_gather(pad_v, [row_vec, col_vec])
                        ptile_v[b, d // 8, j, d % 8, pl.ds(g * 16, 16)] = v
                return carry

            lax.fori_loop(0, 8, g_body, 0)

        def chunk_pair(p, carry):
            for b in range(2):
                i = 2 * p + b
                b_copy(b).wait()
                @pl.when(i + 1 < n)
                def _():
                    a_copy(0, 1 - b).wait()
                    b_copy(1 - b).start()
                @pl.when(i >= 2)
                def _():
                    for d in d_copies(0, b):
                        d.wait()
                transpose(b)
                for d in d_copies(i, b):
                    d.start()
                @pl.when(i + 2 < n)
                def _():
                    a_copy(i + 2, b).start()
            return carry

        a_copy(0, 0).start()
        a_copy(1, 1).start()
        a_copy(0, 0).wait()
        b_copy(0).start()

        lax.fori_loop(0, n // 2, chunk_pair, 0)

        for b in range(2):
            for d in d_copies(0, b):
                d.wait()

    return body, mesh


def kernel(lookup, table):
    batch, hist = lookup.shape
    info = plsc.get_sparse_core_info()
    body, mesh = _gather_kernel(batch, hist, info.num_cores,
                                info.num_subcores)
    p = pl.kernel(
        body,
        out_type=jax.ShapeDtypeStruct(
            (hist, 4, batch // BLK, 8, BLK), jnp.float32),
        mesh=mesh,
        scratch_types=[
            pltpu.VMEM((2, CHUNK), jnp.int32),
            pltpu.VMEM((2, CHUNK, D_FIELD), jnp.float32),
            pltpu.VMEM((CHUNK, D_FIELD + 1), jnp.float32),
            pltpu.VMEM((2, 4, UNITS, 8, BLK), jnp.float32),
        ] + [pltpu.SemaphoreType.DMA] * 6,
        compiler_params=pltpu.CompilerParams(use_tc_tiling_on_sc=False,
                                             needs_layout_passes=False),
    )(lookup.T.astype(jnp.int32), table)
    q = p.transpose(2, 4, 0, 1, 3)
    return q.reshape(batch, hist, D_FIELD)


# speedup vs baseline: 2.7009x; 2.1179x over previous
import jax
import jax.numpy as jnp
from jax import lax
from jax.experimental import pallas as pl
from jax.experimental.pallas import tpu as pltpu
from jax.experimental.pallas import tpu_sc as plsc

D_FIELD = 32
BLK = 128
UNITS = 4
CHUNK = UNITS * BLK


def _gather_kernel(batch, hist, num_cores, num_subcores):
    nw = num_cores * num_subcores
    n_blocks = batch // BLK
    blocks_per_t = n_blocks // UNITS
    n_chunks_total = hist * blocks_per_t
    n = n_chunks_total // nw
    assert batch % (BLK * UNITS) == 0 and n_chunks_total % nw == 0
    assert n % 2 == 0
    mesh = plsc.VectorSubcoreMesh(core_axis_name="c", subcore_axis_name="s")

    def body(lookup_hbm, table_hbm, out_hbm, idx_v, rows_v, pad_v, ptile_v,
             *sems):
        sem_i = sems[0:2]
        sem_g = sems[2:4]
        sem_o = sems[4:6]
        wid = lax.axis_index("s") * num_cores + lax.axis_index("c")
        c0 = wid * n
        lane = jax.lax.iota(jnp.int32, 16)

        def coords(i):
            cg = c0 + i
            return cg // blocks_per_t, (cg % blocks_per_t) * UNITS

        def a_copy(i, b):
            t, bc0 = coords(i)
            return pltpu.make_async_copy(
                lookup_hbm.at[t, pl.ds(bc0 * BLK, CHUNK)], idx_v.at[b],
                sem_i[b])

        def b_copy(b):
            return pltpu.make_async_copy(
                table_hbm.at[idx_v.at[b]], rows_v.at[b], sem_g[b])

        def d_copies(i, b):
            t, bc0 = coords(i)
            return [
                pltpu.make_async_copy(
                    ptile_v.at[b, dr],
                    out_hbm.at[t, dr, pl.ds(bc0, UNITS)], sem_o[b])
                for dr in range(4)
            ]

        def transpose(b):
            def repitch(rr, carry):
                vals = []
                for k in range(8):
                    r = rr * 8 + k
                    vals.append(rows_v[b, r, pl.ds(0, 16)])
                    vals.append(rows_v[b, r, pl.ds(16, 16)])
                for k in range(8):
                    r = rr * 8 + k
                    pad_v[r, pl.ds(0, 16)] = vals[2 * k]
                    pad_v[r, pl.ds(16, 16)] = vals[2 * k + 1]
                return carry

            lax.fori_loop(0, CHUNK // 8, repitch, 0)

            def g_body(g, carry):
                for j in range(UNITS):
                    row_vec = lane + (j * BLK + g * 16)
                    for d0 in range(0, D_FIELD, 8):
                        vs = []
                        for k in range(8):
                            col_vec = jnp.full((16,), d0 + k, jnp.int32)
                            vs.append(
                                plsc.load_gather(pad_v, [row_vec, col_vec]))
                        for k in range(8):
                            d = d0 + k
                            ptile_v[b, d // 8, j, d % 8,
                                    pl.ds(g * 16, 16)] = vs[k]
                return carry

            lax.fori_loop(0, 8, g_body, 0)

        def chunk_pair(p, carry):
            for b in range(2):
                i = 2 * p + b
                b_copy(b).wait()
                @pl.when(i + 1 < n)
                def _():
                    a_copy(0, 1 - b).wait()
                    b_copy(1 - b).start()
                @pl.when(i >= 2)
                def _():
                    for d in d_copies(0, b):
                        d.wait()
                transpose(b)
                for d in d_copies(i, b):
                    d.start()
                @pl.when(i + 2 < n)
                def _():
                    a_copy(i + 2, b).start()
            return carry

        a_copy(0, 0).start()
        a_copy(1, 1).start()
        a_copy(0, 0).wait()
        b_copy(0).start()

        lax.fori_loop(0, n // 2, chunk_pair, 0)

        for b in range(2):
            for d in d_copies(0, b):
                d.wait()

    return body, mesh


def kernel(lookup, table):
    batch, hist = lookup.shape
    info = plsc.get_sparse_core_info()
    body, mesh = _gather_kernel(batch, hist, info.num_cores,
                                info.num_subcores)
    p = pl.kernel(
        body,
        out_type=jax.ShapeDtypeStruct(
            (hist, 4, batch // BLK, 8, BLK), jnp.float32),
        mesh=mesh,
        scratch_types=[
            pltpu.VMEM((2, CHUNK), jnp.int32),
            pltpu.VMEM((2, CHUNK, D_FIELD), jnp.float32),
            pltpu.VMEM((CHUNK, D_FIELD + 1), jnp.float32),
            pltpu.VMEM((2, 4, UNITS, 8, BLK), jnp.float32),
        ] + [pltpu.SemaphoreType.DMA] * 6,
        compiler_params=pltpu.CompilerParams(use_tc_tiling_on_sc=False,
                                             needs_layout_passes=False),
    )(lookup.T.astype(jnp.int32), table)
    q = p.transpose(2, 4, 0, 1, 3)
    return q.reshape(batch, hist, D_FIELD)


# speedup vs baseline: 2.7432x; 1.0157x over previous
import jax
import jax.numpy as jnp
from jax import lax
from jax.experimental import pallas as pl
from jax.experimental.pallas import tpu as pltpu
from jax.experimental.pallas import tpu_sc as plsc

D_FIELD = 32
BLK = 128
UNITS = 4
CHUNK = UNITS * BLK


def _gather_kernel(batch, hist, num_cores, num_subcores):
    nw = num_cores * num_subcores
    n_blocks = batch // BLK
    blocks_per_t = n_blocks // UNITS
    n_chunks_total = hist * blocks_per_t
    n = n_chunks_total // nw
    assert batch % (BLK * UNITS) == 0 and n_chunks_total % nw == 0
    assert n % 2 == 0
    mesh = plsc.VectorSubcoreMesh(core_axis_name="c", subcore_axis_name="s")

    def body(lookup_hbm, table_hbm, out_hbm, idx_v, rows_v, pad_v, ptile_v,
             *sems):
        sem_i = sems[0:2]
        sem_g = sems[2:4]
        sem_o = sems[4:6]
        wid = lax.axis_index("s") * num_cores + lax.axis_index("c")
        c0 = wid * n
        lane = jax.lax.iota(jnp.int32, 16)

        def coords(i):
            cg = c0 + i
            return cg // blocks_per_t, (cg % blocks_per_t) * UNITS

        def a_copy(i, b):
            t, bc0 = coords(i)
            return pltpu.make_async_copy(
                lookup_hbm.at[t, pl.ds(bc0 * BLK, CHUNK)], idx_v.at[b],
                sem_i[b])

        def b_copy(b):
            return pltpu.make_async_copy(
                table_hbm.at[idx_v.at[b]], rows_v.at[b], sem_g[b])

        def d_copies(i, b):
            t, bc0 = coords(i)
            return [
                pltpu.make_async_copy(
                    ptile_v.at[b, dr],
                    out_hbm.at[t, dr, pl.ds(bc0, UNITS)], sem_o[b])
                for dr in range(4)
            ]

        def transpose(b):
            def repitch(rr, carry):
                vals = []
                for k in range(16):
                    r = rr * 16 + k
                    vals.append(rows_v[b, r, pl.ds(0, 16)])
                    vals.append(rows_v[b, r, pl.ds(16, 16)])
                for k in range(16):
                    r = rr * 16 + k
                    pad_v[r, pl.ds(0, 16)] = vals[2 * k]
                    pad_v[r, pl.ds(16, 16)] = vals[2 * k + 1]
                return carry

            lax.fori_loop(0, CHUNK // 16, repitch, 0)

            def g_body(g, carry):
                for j in range(UNITS):
                    row_vec = lane + (j * BLK + g * 16)
                    for d0 in range(0, D_FIELD, 16):
                        vs = []
                        for k in range(16):
                            col_vec = jnp.full((16,), d0 + k, jnp.int32)
                            vs.append(
                                plsc.load_gather(pad_v, [row_vec, col_vec]))
                        for k in range(16):
                            d = d0 + k
                            ptile_v[b, d // 8, j, d % 8,
                                    pl.ds(g * 16, 16)] = vs[k]
                return carry

            lax.fori_loop(0, 8, g_body, 0)

        def chunk_pair(p, carry):
            for b in range(2):
                i = 2 * p + b
                b_copy(b).wait()
                @pl.when(i + 1 < n)
                def _():
                    a_copy(0, 1 - b).wait()
                    b_copy(1 - b).start()
                @pl.when(i >= 2)
                def _():
                    for d in d_copies(0, b):
                        d.wait()
                transpose(b)
                for d in d_copies(i, b):
                    d.start()
                @pl.when(i + 2 < n)
                def _():
                    a_copy(i + 2, b).start()
            return carry

        a_copy(0, 0).start()
        a_copy(1, 1).start()
        a_copy(0, 0).wait()
        b_copy(0).start()

        lax.fori_loop(0, n // 2, chunk_pair, 0)

        for b in range(2):
            for d in d_copies(0, b):
                d.wait()

    return body, mesh


def kernel(lookup, table):
    batch, hist = lookup.shape
    info = plsc.get_sparse_core_info()
    body, mesh = _gather_kernel(batch, hist, info.num_cores,
                                info.num_subcores)
    p = pl.kernel(
        body,
        out_type=jax.ShapeDtypeStruct(
            (hist, 4, batch // BLK, 8, BLK), jnp.float32),
        mesh=mesh,
        scratch_types=[
            pltpu.VMEM((2, CHUNK), jnp.int32),
            pltpu.VMEM((2, CHUNK, D_FIELD), jnp.float32),
            pltpu.VMEM((CHUNK, D_FIELD + 1), jnp.float32),
            pltpu.VMEM((2, 4, UNITS, 8, BLK), jnp.float32),
        ] + [pltpu.SemaphoreType.DMA] * 6,
        compiler_params=pltpu.CompilerParams(use_tc_tiling_on_sc=False,
                                             needs_layout_passes=False),
    )(lookup.T.astype(jnp.int32), table)
    q = p.transpose(2, 4, 0, 1, 3)
    return q.reshape(batch, hist, D_FIELD)


# speedup vs baseline: 3.1779x; 1.1584x over previous
import jax
import jax.numpy as jnp
from jax import lax
from jax.experimental import pallas as pl
from jax.experimental.pallas import tpu as pltpu
from jax.experimental.pallas import tpu_sc as plsc

D_FIELD = 32
BLK = 128
UNITS = 4
CHUNK = UNITS * BLK


def _gather_kernel(batch, hist, num_cores, num_subcores):
    nw = num_cores * num_subcores
    n_blocks = batch // BLK
    blocks_per_t = n_blocks // UNITS
    n_chunks_total = hist * blocks_per_t
    n = n_chunks_total // nw
    assert batch % (BLK * UNITS) == 0 and n_chunks_total % nw == 0
    assert n % 2 == 0
    mesh = plsc.VectorSubcoreMesh(core_axis_name="c", subcore_axis_name="s")

    def body(lookup_hbm, table_hbm, out_hbm, idx_v, rows_v, pad_v, ptile_v,
             *sems):
        sem_i = sems[0:2]
        sem_g = sems[2:4]
        sem_o = sems[4:6]
        wid = lax.axis_index("s") * num_cores + lax.axis_index("c")
        c0 = wid * n
        lane = jax.lax.iota(jnp.int32, 16)

        def coords(i):
            cg = c0 + i
            return cg // blocks_per_t, (cg % blocks_per_t) * UNITS

        def a_copy(i, b):
            t, bc0 = coords(i)
            return pltpu.make_async_copy(
                lookup_hbm.at[t, pl.ds(bc0 * BLK, CHUNK)], idx_v.at[b],
                sem_i[b])

        def b_copy(b):
            return pltpu.make_async_copy(
                table_hbm.at[idx_v.at[b]], rows_v.at[b], sem_g[b])

        def d_copies(i, b):
            t, bc0 = coords(i)
            return [
                pltpu.make_async_copy(
                    ptile_v.at[b, dr],
                    out_hbm.at[t, dr, pl.ds(bc0, UNITS)], sem_o[b])
                for dr in range(4)
            ]

        def transpose(b):
            def repitch(rr, carry):
                vals = {}
                for k in range(32 + 8):
                    if k < 32:
                        r = rr * 16 + k // 2
                        off = (k % 2) * 16
                        vals[k] = rows_v[b, r, pl.ds(off, 16)]
                    if k >= 8:
                        m = k - 8
                        r = rr * 16 + m // 2
                        off = (m % 2) * 16
                        pad_v[r, pl.ds(off, 16)] = vals[m]
                return carry

            lax.fori_loop(0, CHUNK // 16, repitch, 0)

            def g_body(g, carry):
                for j in range(UNITS):
                    row_vec = lane + (j * BLK + g * 16)
                    vs = {}
                    for k in range(D_FIELD + 8):
                        if k < D_FIELD:
                            col_vec = jnp.full((16,), k, jnp.int32)
                            vs[k] = plsc.load_gather(pad_v,
                                                     [row_vec, col_vec])
                        if k >= 8:
                            d = k - 8
                            ptile_v[b, d // 8, j, d % 8,
                                    pl.ds(g * 16, 16)] = vs[d]
                return carry

            lax.fori_loop(0, 8, g_body, 0)

        def chunk_pair(p, carry):
            for b in range(2):
                i = 2 * p + b
                b_copy(b).wait()
                @pl.when(i + 1 < n)
                def _():
                    a_copy(0, 1 - b).wait()
                    b_copy(1 - b).start()
                @pl.when(i >= 2)
                def _():
                    for d in d_copies(0, b):
                        d.wait()
                transpose(b)
                for d in d_copies(i, b):
                    d.start()
                @pl.when(i + 2 < n)
                def _():
                    a_copy(i + 2, b).start()
            return carry

        a_copy(0, 0).start()
        a_copy(1, 1).start()
        a_copy(0, 0).wait()
        b_copy(0).start()

        lax.fori_loop(0, n // 2, chunk_pair, 0)

        for b in range(2):
            for d in d_copies(0, b):
                d.wait()

    return body, mesh


def kernel(lookup, table):
    batch, hist = lookup.shape
    info = plsc.get_sparse_core_info()
    body, mesh = _gather_kernel(batch, hist, info.num_cores,
                                info.num_subcores)
    p = pl.kernel(
        body,
        out_type=jax.ShapeDtypeStruct(
            (hist, 4, batch // BLK, 8, BLK), jnp.float32),
        mesh=mesh,
        scratch_types=[
            pltpu.VMEM((2, CHUNK), jnp.int32),
            pltpu.VMEM((2, CHUNK, D_FIELD), jnp.float32),
            pltpu.VMEM((CHUNK, D_FIELD + 1), jnp.float32),
            pltpu.VMEM((2, 4, UNITS, 8, BLK), jnp.float32),
        ] + [pltpu.SemaphoreType.DMA] * 6,
        compiler_params=pltpu.CompilerParams(use_tc_tiling_on_sc=False,
                                             needs_layout_passes=False),
    )(lookup.T.astype(jnp.int32), table)
    q = p.transpose(2, 4, 0, 1, 3)
    return q.reshape(batch, hist, D_FIELD)
